# Initial kernel scaffold; baseline (speedup 1.0000x reference)
import jax
import jax.numpy as jnp
from jax.experimental import pallas as pl


def kernel(x_ap, x_user, x_target, edge_index_serves, edge_index_senses, params):
    raise NotImplementedError("write your pallas kernel here")


# speedup vs baseline: 6.0407x; 6.0407x over previous
import functools

import jax
import jax.numpy as jnp
from jax import lax
from jax.experimental import pallas as pl
from jax.experimental.pallas import tpu as pltpu
from jax.experimental.pallas import tpu_sc as plsc

N = 50000
NP = 50176
E = 600000
EP = 600064
NCH = EP // 128
H = 64
HH = 32
NC, NS = 2, 16
TPT = NP // NS
ZROWS = 448
BR = 1024
GRID = NP // BR

_MESH = plsc.VectorSubcoreMesh(
    core_axis_name="c", subcore_axis_name="s", num_cores=NC, num_subcores=NS)
_SC_PARAMS = pltpu.CompilerParams(use_tc_tiling_on_sc=False, needs_layout_passes=False)


def _leaky(x):
    return jnp.where(x >= 0, x, 0.01 * x)


def _sc_count_body(ei_ref, out_a_ref, out_b_ref, idx_a, idx_b, acc_a, acc_b):
    c = lax.axis_index("c")
    s = lax.axis_index("s")
    wid = s * NC + c

    def zero(i, _):
        zv = jnp.zeros((16,), jnp.float32)
        acc_a[pl.ds(i * 16, 16)] = zv
        acc_b[pl.ds(i * 16, 16)] = zv
        return 0
    lax.fori_loop(0, NP // 16, zero, 0)

    ones = jnp.ones((16,), jnp.float32)
    n_t = (NCH - wid + 31) // 32

    def body(i, _):
        ch = wid + i * 32
        pltpu.sync_copy(ei_ref.at[0, ch], idx_a)
        pltpu.sync_copy(ei_ref.at[1, ch], idx_b)
        for k in range(8):
            plsc.addupdate_scatter(acc_a, [idx_a[pl.ds(k * 16, 16)]], ones)
            plsc.addupdate_scatter(acc_b, [idx_b[pl.ds(k * 16, 16)]], ones)
        return 0
    lax.fori_loop(0, n_t, body, 0)

    pltpu.sync_copy(acc_a, out_a_ref.at[wid])
    pltpu.sync_copy(acc_b, out_b_ref.at[wid])


_sc_count = pl.kernel(
    _sc_count_body,
    out_type=(jax.ShapeDtypeStruct((NC * NS, NP), jnp.float32),
              jax.ShapeDtypeStruct((NC * NS, NP), jnp.float32)),
    mesh=_MESH,
    compiler_params=_SC_PARAMS,
    scratch_types=[
        pltpu.VMEM((128,), jnp.int32),
        pltpu.VMEM((128,), jnp.int32),
        pltpu.VMEM((NP,), jnp.float32),
        pltpu.VMEM((NP,), jnp.float32),
    ],
)


def _sc_aggr_body(src_row, dst_row,
                  tlo_ref, thi_ref, ei_ref, slo_ref, shi_ref,
                  idx_s, idx_d, rows, zbuf, acc, sem):
    c = lax.axis_index("c")
    s = lax.axis_index("s")

    def zrow(i, _):
        zv = jnp.zeros((16,), jnp.float32)
        zbuf[i, pl.ds(0, 16)] = zv
        zbuf[i, pl.ds(16, 16)] = zv
        return 0
    lax.fori_loop(0, ZROWS, zrow, 0)

    r0 = s * TPT
    for j in range(TPT // ZROWS):
        pltpu.sync_copy(zbuf, acc.at[pl.ds(r0 + j * ZROWS, ZROWS)])
    plsc.subcore_barrier()

    def body(i, _):
        ch = s + i * NS
        pltpu.sync_copy(ei_ref.at[src_row, ch], idx_s)
        pltpu.sync_copy(ei_ref.at[dst_row, ch], idx_d)

        @pl.when(c == 0)
        def _():
            pltpu.async_copy(tlo_ref.at[idx_s], rows, sem).wait()

        @pl.when(c == 1)
        def _():
            pltpu.async_copy(thi_ref.at[idx_s], rows, sem).wait()

        pltpu.sync_copy(rows, acc.at[idx_d], add=True)
        return 0
    lax.fori_loop(0, NCH // NS, body, 0)
    plsc.subcore_barrier()

    @pl.when(c == 0)
    def _():
        pltpu.sync_copy(acc.at[pl.ds(r0, TPT)], slo_ref.at[pl.ds(r0, TPT)])

    @pl.when(c == 1)
    def _():
        pltpu.sync_copy(acc.at[pl.ds(r0, TPT)], shi_ref.at[pl.ds(r0, TPT)])


def _make_sc_aggr(src_row, dst_row):
    return pl.kernel(
        functools.partial(_sc_aggr_body, src_row, dst_row),
        out_type=(jax.ShapeDtypeStruct((NP, HH), jnp.float32),
                  jax.ShapeDtypeStruct((NP, HH), jnp.float32)),
        mesh=_MESH,
        compiler_params=_SC_PARAMS,
        scratch_types=[
            pltpu.VMEM((128,), jnp.int32),
            pltpu.VMEM((128,), jnp.int32),
            pltpu.VMEM((128, HH), jnp.float32),
            pltpu.VMEM((ZROWS, HH), jnp.float32),
            pltpu.VMEM_SHARED((NP, HH), jnp.float32),
            pltpu.SemaphoreType.DMA,
        ],
    )


_sc_aggr_fwd = _make_sc_aggr(0, 1)
_sc_aggr_rev = _make_sc_aggr(1, 0)


def _sc_edge_head_body(ta_ref, tb_ref, ei_ref, out_ref,
                       tsa, tsb, idx_s, idx_d, ob):
    c = lax.axis_index("c")
    s = lax.axis_index("s")
    wid = s * NC + c
    pltpu.sync_copy(ta_ref, tsa)
    pltpu.sync_copy(tb_ref, tsb)
    n_t = (NCH - wid + 31) // 32

    def body(i, _):
        ch = wid + i * 32
        pltpu.sync_copy(ei_ref.at[0, ch], idx_s)
        pltpu.sync_copy(ei_ref.at[1, ch], idx_d)
        for k in range(8):
            v = (plsc.load_gather(tsa, [idx_s[pl.ds(k * 16, 16)]])
                 + plsc.load_gather(tsb, [idx_d[pl.ds(k * 16, 16)]]))
            ob[pl.ds(k * 16, 16)] = v
        pltpu.sync_copy(ob, out_ref.at[ch])
        return 0
    lax.fori_loop(0, n_t, body, 0)


_sc_edge_head = pl.kernel(
    _sc_edge_head_body,
    out_type=jax.ShapeDtypeStruct((NCH, 128), jnp.float32),
    mesh=_MESH,
    compiler_params=_SC_PARAMS,
    scratch_types=[
        pltpu.VMEM((NP,), jnp.float32),
        pltpu.VMEM((NP,), jnp.float32),
        pltpu.VMEM((128,), jnp.int32),
        pltpu.VMEM((128,), jnp.int32),
        pltpu.VMEM((128,), jnp.float32),
    ],
)


def _tc_proj_body(xa, wa, ba, xu, wu, bu, xt, wt, bt,
                  oal, oah, oul, ouh, otl, oth):
    def one(x, w, b, ol, oh):
        h = jnp.dot(x[...], w[...], preferred_element_type=jnp.float32) + b[...]
        ol[...] = h[:, :HH]
        oh[...] = h[:, HH:]
    one(xa, wa, ba, oal, oah)
    one(xu, wu, bu, oul, ouh)
    one(xt, wt, bt, otl, oth)


def _rows_spec(w):
    return pl.BlockSpec((BR, w), lambda i: (i, 0))


def _full_spec(r, c):
    return pl.BlockSpec((r, c), lambda i: (0, 0))


_tc_proj = pl.pallas_call(
    _tc_proj_body,
    grid=(GRID,),
    in_specs=[_rows_spec(128), _full_spec(128, H), _full_spec(1, H)] * 3,
    out_specs=[_rows_spec(HH)] * 6,
    out_shape=[jax.ShapeDtypeStruct((NP, HH), jnp.float32)] * 6,
)


def _tc_upd1_body(sl, sh, cp, xl, xh, wl, bl, wr, ol, oh):
    S = jnp.concatenate([sl[...], sh[...]], axis=1)
    cnt = jnp.sum(cp[...], axis=0)
    mean = S * (1.0 / jnp.maximum(cnt, 1.0))[:, None]
    X = jnp.concatenate([xl[...], xh[...]], axis=1)
    y = (jnp.dot(mean, wl[...], preferred_element_type=jnp.float32) + bl[...]
         + jnp.dot(X, wr[...], preferred_element_type=jnp.float32))
    y = _leaky(y)
    ol[...] = y[:, :HH]
    oh[...] = y[:, HH:]


_tc_upd1 = pl.pallas_call(
    _tc_upd1_body,
    grid=(GRID,),
    in_specs=[_rows_spec(HH), _rows_spec(HH),
              pl.BlockSpec((NC * NS, BR), lambda i: (0, i)),
              _rows_spec(HH), _rows_spec(HH),
              _full_spec(H, H), _full_spec(1, H), _full_spec(H, H)],
    out_specs=[_rows_spec(HH)] * 2,
    out_shape=[jax.ShapeDtypeStruct((NP, HH), jnp.float32)] * 2,
)


def _tc_upd2_body(s1l, s1h, cp1, s2l, s2h, cp2, xl, xh,
                  wl1, bl1, wr1, wl2, bl2, wr2, ol, oh):
    S1 = jnp.concatenate([s1l[...], s1h[...]], axis=1)
    S2 = jnp.concatenate([s2l[...], s2h[...]], axis=1)
    c1 = jnp.sum(cp1[...], axis=0)
    c2 = jnp.sum(cp2[...], axis=0)
    m1 = S1 * (1.0 / jnp.maximum(c1, 1.0))[:, None]
    m2 = S2 * (1.0 / jnp.maximum(c2, 1.0))[:, None]
    X = jnp.concatenate([xl[...], xh[...]], axis=1)
    y = (jnp.dot(m1, wl1[...], preferred_element_type=jnp.float32) + bl1[...]
         + jnp.dot(m2, wl2[...], preferred_element_type=jnp.float32) + bl2[...]
         + jnp.dot(X, wr1[...] + wr2[...], preferred_element_type=jnp.float32))
    y = _leaky(y)
    ol[...] = y[:, :HH]
    oh[...] = y[:, HH:]


_tc_upd2 = pl.pallas_call(
    _tc_upd2_body,
    grid=(GRID,),
    in_specs=[_rows_spec(HH), _rows_spec(HH),
              pl.BlockSpec((NC * NS, BR), lambda i: (0, i)),
              _rows_spec(HH), _rows_spec(HH),
              pl.BlockSpec((NC * NS, BR), lambda i: (0, i)),
              _rows_spec(HH), _rows_spec(HH),
              _full_spec(H, H), _full_spec(1, H), _full_spec(H, H),
              _full_spec(H, H), _full_spec(1, H), _full_spec(H, H)],
    out_specs=[_rows_spec(HH)] * 2,
    out_shape=[jax.ShapeDtypeStruct((NP, HH), jnp.float32)] * 2,
)


def _tc_heads_body(al, ah, ul, uh, tl, th, wa, ba, wu, wt, bt,
                   oa, ou, ot):
    A = jnp.concatenate([al[...], ah[...]], axis=1)
    U = jnp.concatenate([ul[...], uh[...]], axis=1)
    T = jnp.concatenate([tl[...], th[...]], axis=1)
    oa[...] = jnp.dot(A, wa[...], preferred_element_type=jnp.float32) + ba[...]
    ou[...] = jnp.dot(U, wu[...], preferred_element_type=jnp.float32)
    ot[...] = jnp.dot(T, wt[...], preferred_element_type=jnp.float32) + bt[...]


_tc_heads = pl.pallas_call(
    _tc_heads_body,
    grid=(GRID,),
    in_specs=[_rows_spec(HH)] * 6 + [
        _full_spec(H, 4), _full_spec(1, 4),
        _full_spec(H, 1),
        _full_spec(H, 3), _full_spec(1, 3)],
    out_specs=[_rows_spec(4), _rows_spec(1), _rows_spec(3)],
    out_shape=[jax.ShapeDtypeStruct((NP, 4), jnp.float32),
               jax.ShapeDtypeStruct((NP, 1), jnp.float32),
               jax.ShapeDtypeStruct((NP, 3), jnp.float32)],
)


def kernel(x_ap, x_user, x_target, edge_index_serves, edge_index_senses, params):
    p = params
    f32 = jnp.float32

    def pad_edges(ei):
        padc = jnp.full((2, EP - E), N, jnp.int32)
        return jnp.concatenate([ei.astype(jnp.int32), padc], axis=1).reshape(
            2, NCH, 128)

    ei_sv = pad_edges(edge_index_serves)
    ei_sn = pad_edges(edge_index_senses)

    def pad_rows(x):
        return jnp.pad(x, ((0, NP - N), (0, 0)))

    xa, xu, xt = pad_rows(x_ap), pad_rows(x_user), pad_rows(x_target)

    cnt_ap_sv_p, cnt_user_p = _sc_count(ei_sv)
    cnt_ap_sn_p, cnt_tgt_p = _sc_count(ei_sn)

    b2 = lambda b: b.reshape(1, H)
    h_ap = _tc_proj(xa, p['proj_ap_W'], b2(p['proj_ap_b']),
                    xu, p['proj_user_W'], b2(p['proj_user_b']),
                    xt, p['proj_target_W'], b2(p['proj_target_b']))
    hal, hah, hul, huh, htl, hth = h_ap

    for layer in ('c1', 'c2'):
        g = lambda n: p[layer + '_' + n]
        s_sv = _sc_aggr_fwd(hal, hah, ei_sv)
        s_sn = _sc_aggr_fwd(hal, hah, ei_sn)
        s_rsv = _sc_aggr_rev(hul, huh, ei_sv)
        s_rsn = _sc_aggr_rev(htl, hth, ei_sn)
        nul, nuh = _tc_upd1(s_sv[0], s_sv[1], cnt_user_p, hul, huh,
                            g('serves_Wl'), b2(g('serves_bl')), g('serves_Wr'))
        ntl, nth = _tc_upd1(s_sn[0], s_sn[1], cnt_tgt_p, htl, hth,
                            g('senses_Wl'), b2(g('senses_bl')), g('senses_Wr'))
        nal, nah = _tc_upd2(s_rsv[0], s_rsv[1], cnt_ap_sv_p,
                            s_rsn[0], s_rsn[1], cnt_ap_sn_p, hal, hah,
                            g('rev_serves_Wl'), b2(g('rev_serves_bl')),
                            g('rev_serves_Wr'),
                            g('rev_senses_Wl'), b2(g('rev_senses_bl')),
                            g('rev_senses_Wr'))
        hal, hah, hul, huh, htl, hth = nal, nah, nul, nuh, ntl, nth

    wa = jnp.concatenate([p['head_tau_W'], p['head_x_W'][:H],
                          p['head_ytx_W'][:H], p['head_yrx_W'][:H]], axis=1)
    ba = jnp.stack([p['head_tau_b'][0], p['head_x_b'][0],
                    p['head_ytx_b'][0], p['head_yrx_b'][0]]).reshape(1, 4)
    wu = p['head_x_W'][H:]
    wt = jnp.concatenate([p['head_s_W'], p['head_ytx_W'][H:],
                          p['head_yrx_W'][H:]], axis=1)
    bt = jnp.stack([p['head_s_b'][0], jnp.zeros((), f32),
                    jnp.zeros((), f32)]).reshape(1, 3)
    A, U, T = _tc_heads(hal, hah, hul, huh, htl, hth, wa, ba, wu, wt, bt)

    tau = A[:N, 0]
    s_out = T[:N, 0]
    x_log = _sc_edge_head(jnp.ravel(A[:, 1]), jnp.ravel(U[:, 0]),
                          ei_sv).reshape(EP)[:E]
    ytx = _sc_edge_head(jnp.ravel(A[:, 2]), jnp.ravel(T[:, 1]),
                        ei_sn).reshape(EP)[:E]
    yrx = _sc_edge_head(jnp.ravel(A[:, 3]), jnp.ravel(T[:, 2]),
                        ei_sn).reshape(EP)[:E]
    return (tau, s_out, x_log, ytx, yrx)


# speedup vs baseline: 9.5750x; 1.5851x over previous
import functools

import jax
import jax.numpy as jnp
from jax import lax
from jax.experimental import pallas as pl
from jax.experimental.pallas import tpu as pltpu
from jax.experimental.pallas import tpu_sc as plsc

N = 50000
NP = 50176
E = 600000
EP = 610304
NCH = EP // 128
CPT = NCH // 16
KSB = 2
NSB = CPT // KSB
H = 64
HH = 32
NC, NS = 2, 16
TPT = NP // NS
ZROWS = 224
BR = 1024
GRID = NP // BR

_MESH = plsc.VectorSubcoreMesh(
    core_axis_name="c", subcore_axis_name="s", num_cores=NC, num_subcores=NS)
_SC_PARAMS = pltpu.CompilerParams(use_tc_tiling_on_sc=False, needs_layout_passes=False)


def _leaky(x):
    return jnp.where(x >= 0, x, 0.01 * x)


def _sc_count_body(ei_ref, out_a_ref, out_b_ref, idx_a, idx_b, acc_a, acc_b):
    c = lax.axis_index("c")
    s = lax.axis_index("s")
    wid = s * NC + c

    def zero(i, _):
        zv = jnp.zeros((16,), jnp.float32)
        acc_a[pl.ds(i * 16, 16)] = zv
        acc_b[pl.ds(i * 16, 16)] = zv
        return 0
    lax.fori_loop(0, NP // 16, zero, 0)

    ones = jnp.ones((16,), jnp.float32)
    n_t = (NCH - wid + 31) // 32

    def body(i, _):
        ch = wid + i * 32
        pltpu.sync_copy(ei_ref.at[0, ch], idx_a)
        pltpu.sync_copy(ei_ref.at[1, ch], idx_b)
        for k in range(8):
            plsc.addupdate_scatter(acc_a, [idx_a[pl.ds(k * 16, 16)]], ones)
            plsc.addupdate_scatter(acc_b, [idx_b[pl.ds(k * 16, 16)]], ones)
        return 0
    lax.fori_loop(0, n_t, body, 0)

    pltpu.sync_copy(acc_a, out_a_ref.at[wid])
    pltpu.sync_copy(acc_b, out_b_ref.at[wid])


_sc_count = pl.kernel(
    _sc_count_body,
    out_type=(jax.ShapeDtypeStruct((NC * NS, NP), jnp.float32),
              jax.ShapeDtypeStruct((NC * NS, NP), jnp.float32)),
    mesh=_MESH,
    compiler_params=_SC_PARAMS,
    scratch_types=[
        pltpu.VMEM((128,), jnp.int32),
        pltpu.VMEM((128,), jnp.int32),
        pltpu.VMEM((NP,), jnp.float32),
        pltpu.VMEM((NP,), jnp.float32),
    ],
)


def _sc_aggr_body(src_row, dst_row,
                  tlo_ref, thi_ref, ei_ref, slo_ref, shi_ref,
                  ixs_a, ixd_a, ixs_b, ixd_b, rows_a, rows_b, zbuf, acc,
                  sia, sib, sga, sgb, ssa, ssb):
    c = lax.axis_index("c")
    s = lax.axis_index("s")

    def zrow(i, _):
        zv = jnp.zeros((16,), jnp.float32)
        zbuf[i, pl.ds(0, 16)] = zv
        zbuf[i, pl.ds(16, 16)] = zv
        return 0
    lax.fori_loop(0, ZROWS, zrow, 0)

    r0 = s * TPT
    for j in range(TPT // ZROWS):
        pltpu.sync_copy(zbuf, acc.at[pl.ds(r0 + j * ZROWS, ZROWS)])
    plsc.subcore_barrier()

    t0 = s * CPT

    def fire_idx(b, ixs, ixd, sem):
        pltpu.async_copy(ei_ref.at[src_row, pl.ds(t0 + b * KSB, KSB)], ixs, sem)
        pltpu.async_copy(ei_ref.at[dst_row, pl.ds(t0 + b * KSB, KSB)], ixd, sem)

    def wait_idx(ixs, ixd, sem):
        pltpu.make_async_copy(ei_ref.at[src_row, pl.ds(0, KSB)], ixs, sem).wait()
        pltpu.make_async_copy(ei_ref.at[dst_row, pl.ds(0, KSB)], ixd, sem).wait()

    def fire_g(ixs, rows, sem):
        @pl.when(c == 0)
        def _():
            for j in range(KSB):
                pltpu.async_copy(tlo_ref.at[ixs.at[j]], rows.at[j], sem)

        @pl.when(c == 1)
        def _():
            for j in range(KSB):
                pltpu.async_copy(thi_ref.at[ixs.at[j]], rows.at[j], sem)

    def drain_g(ixs, rows, sem):
        for j in range(KSB):
            pltpu.make_async_copy(tlo_ref.at[ixs.at[j]], rows.at[j], sem).wait()

    def fire_s(ixd, rows, sem):
        for j in range(KSB):
            pltpu.async_copy(rows.at[j], acc.at[ixd.at[j]], sem, add=True)

    def drain_s(ixd, rows, sem):
        for j in range(KSB):
            pltpu.make_async_copy(rows.at[j], acc.at[ixd.at[j]], sem).wait()

    A = (ixs_a, ixd_a, rows_a)
    B = (ixs_b, ixd_b, rows_b)

    fire_idx(0, ixs_a, ixd_a, sia)
    wait_idx(ixs_a, ixd_a, sia)
    fire_g(ixs_a, rows_a, sga)

    def pair(i, _):
        b1 = 2 * i + 1
        b2 = 2 * i + 2
        fire_idx(b1, ixs_b, ixd_b, sib)
        drain_g(ixs_a, rows_a, sga)
        fire_s(ixd_a, rows_a, ssa)
        wait_idx(ixs_b, ixd_b, sib)
        fire_g(ixs_b, rows_b, sgb)
        drain_s(ixd_a, rows_a, ssa)
        fire_idx(b2, ixs_a, ixd_a, sia)
        drain_g(ixs_b, rows_b, sgb)
        fire_s(ixd_b, rows_b, ssb)
        wait_idx(ixs_a, ixd_a, sia)
        fire_g(ixs_a, rows_a, sga)
        drain_s(ixd_b, rows_b, ssb)
        return 0
    lax.fori_loop(0, (NSB - 1) // 2, pair, 0)

    drain_g(ixs_a, rows_a, sga)
    fire_s(ixd_a, rows_a, ssa)
    drain_s(ixd_a, rows_a, ssa)

    plsc.subcore_barrier()

    @pl.when(c == 0)
    def _():
        pltpu.sync_copy(acc.at[pl.ds(r0, TPT)], slo_ref.at[pl.ds(r0, TPT)])

    @pl.when(c == 1)
    def _():
        pltpu.sync_copy(acc.at[pl.ds(r0, TPT)], shi_ref.at[pl.ds(r0, TPT)])


def _make_sc_aggr(src_row, dst_row):
    return pl.kernel(
        functools.partial(_sc_aggr_body, src_row, dst_row),
        out_type=(jax.ShapeDtypeStruct((NP, HH), jnp.float32),
                  jax.ShapeDtypeStruct((NP, HH), jnp.float32)),
        mesh=_MESH,
        compiler_params=_SC_PARAMS,
        scratch_types=[
            pltpu.VMEM((KSB, 128), jnp.int32),
            pltpu.VMEM((KSB, 128), jnp.int32),
            pltpu.VMEM((KSB, 128), jnp.int32),
            pltpu.VMEM((KSB, 128), jnp.int32),
            pltpu.VMEM((KSB, 128, HH), jnp.float32),
            pltpu.VMEM((KSB, 128, HH), jnp.float32),
            pltpu.VMEM((ZROWS, HH), jnp.float32),
            pltpu.VMEM_SHARED((NP, HH), jnp.float32),
            pltpu.SemaphoreType.DMA,
            pltpu.SemaphoreType.DMA,
            pltpu.SemaphoreType.DMA,
            pltpu.SemaphoreType.DMA,
            pltpu.SemaphoreType.DMA,
            pltpu.SemaphoreType.DMA,
        ],
    )


_sc_aggr_fwd = _make_sc_aggr(0, 1)
_sc_aggr_rev = _make_sc_aggr(1, 0)


def _sc_edge_head_body(ta_ref, tb_ref, ei_ref, out_ref,
                       tsa, tsb, idx_s, idx_d, ob):
    c = lax.axis_index("c")
    s = lax.axis_index("s")
    wid = s * NC + c
    pltpu.sync_copy(ta_ref, tsa)
    pltpu.sync_copy(tb_ref, tsb)
    n_t = (NCH - wid + 31) // 32

    def body(i, _):
        ch = wid + i * 32
        pltpu.sync_copy(ei_ref.at[0, ch], idx_s)
        pltpu.sync_copy(ei_ref.at[1, ch], idx_d)
        for k in range(8):
            v = (plsc.load_gather(tsa, [idx_s[pl.ds(k * 16, 16)]])
                 + plsc.load_gather(tsb, [idx_d[pl.ds(k * 16, 16)]]))
            ob[pl.ds(k * 16, 16)] = v
        pltpu.sync_copy(ob, out_ref.at[ch])
        return 0
    lax.fori_loop(0, n_t, body, 0)


_sc_edge_head = pl.kernel(
    _sc_edge_head_body,
    out_type=jax.ShapeDtypeStruct((NCH, 128), jnp.float32),
    mesh=_MESH,
    compiler_params=_SC_PARAMS,
    scratch_types=[
        pltpu.VMEM((NP,), jnp.float32),
        pltpu.VMEM((NP,), jnp.float32),
        pltpu.VMEM((128,), jnp.int32),
        pltpu.VMEM((128,), jnp.int32),
        pltpu.VMEM((128,), jnp.float32),
    ],
)


def _tc_proj_body(xa, wa, ba, xu, wu, bu, xt, wt, bt,
                  oal, oah, oul, ouh, otl, oth):
    def one(x, w, b, ol, oh):
        h = jnp.dot(x[...], w[...], preferred_element_type=jnp.float32) + b[...]
        ol[...] = h[:, :HH]
        oh[...] = h[:, HH:]
    one(xa, wa, ba, oal, oah)
    one(xu, wu, bu, oul, ouh)
    one(xt, wt, bt, otl, oth)


def _rows_spec(w):
    return pl.BlockSpec((BR, w), lambda i: (i, 0))


def _full_spec(r, c):
    return pl.BlockSpec((r, c), lambda i: (0, 0))


_tc_proj = pl.pallas_call(
    _tc_proj_body,
    grid=(GRID,),
    in_specs=[_rows_spec(128), _full_spec(128, H), _full_spec(1, H)] * 3,
    out_specs=[_rows_spec(HH)] * 6,
    out_shape=[jax.ShapeDtypeStruct((NP, HH), jnp.float32)] * 6,
)


def _tc_upd1_body(sl, sh, cp, xl, xh, wl, bl, wr, ol, oh):
    S = jnp.concatenate([sl[...], sh[...]], axis=1)
    cnt = jnp.sum(cp[...], axis=0)
    mean = S * (1.0 / jnp.maximum(cnt, 1.0))[:, None]
    X = jnp.concatenate([xl[...], xh[...]], axis=1)
    y = (jnp.dot(mean, wl[...], preferred_element_type=jnp.float32) + bl[...]
         + jnp.dot(X, wr[...], preferred_element_type=jnp.float32))
    y = _leaky(y)
    ol[...] = y[:, :HH]
    oh[...] = y[:, HH:]


_tc_upd1 = pl.pallas_call(
    _tc_upd1_body,
    grid=(GRID,),
    in_specs=[_rows_spec(HH), _rows_spec(HH),
              pl.BlockSpec((NC * NS, BR), lambda i: (0, i)),
              _rows_spec(HH), _rows_spec(HH),
              _full_spec(H, H), _full_spec(1, H), _full_spec(H, H)],
    out_specs=[_rows_spec(HH)] * 2,
    out_shape=[jax.ShapeDtypeStruct((NP, HH), jnp.float32)] * 2,
)


def _tc_upd2_body(s1l, s1h, cp1, s2l, s2h, cp2, xl, xh,
                  wl1, bl1, wr1, wl2, bl2, wr2, ol, oh):
    S1 = jnp.concatenate([s1l[...], s1h[...]], axis=1)
    S2 = jnp.concatenate([s2l[...], s2h[...]], axis=1)
    c1 = jnp.sum(cp1[...], axis=0)
    c2 = jnp.sum(cp2[...], axis=0)
    m1 = S1 * (1.0 / jnp.maximum(c1, 1.0))[:, None]
    m2 = S2 * (1.0 / jnp.maximum(c2, 1.0))[:, None]
    X = jnp.concatenate([xl[...], xh[...]], axis=1)
    y = (jnp.dot(m1, wl1[...], preferred_element_type=jnp.float32) + bl1[...]
         + jnp.dot(m2, wl2[...], preferred_element_type=jnp.float32) + bl2[...]
         + jnp.dot(X, wr1[...] + wr2[...], preferred_element_type=jnp.float32))
    y = _leaky(y)
    ol[...] = y[:, :HH]
    oh[...] = y[:, HH:]


_tc_upd2 = pl.pallas_call(
    _tc_upd2_body,
    grid=(GRID,),
    in_specs=[_rows_spec(HH), _rows_spec(HH),
              pl.BlockSpec((NC * NS, BR), lambda i: (0, i)),
              _rows_spec(HH), _rows_spec(HH),
              pl.BlockSpec((NC * NS, BR), lambda i: (0, i)),
              _rows_spec(HH), _rows_spec(HH),
              _full_spec(H, H), _full_spec(1, H), _full_spec(H, H),
              _full_spec(H, H), _full_spec(1, H), _full_spec(H, H)],
    out_specs=[_rows_spec(HH)] * 2,
    out_shape=[jax.ShapeDtypeStruct((NP, HH), jnp.float32)] * 2,
)


def _tc_heads_body(al, ah, ul, uh, tl, th, wa, ba, wu, wt, bt,
                   oa, ou, ot):
    A = jnp.concatenate([al[...], ah[...]], axis=1)
    U = jnp.concatenate([ul[...], uh[...]], axis=1)
    T = jnp.concatenate([tl[...], th[...]], axis=1)
    oa[...] = jnp.dot(A, wa[...], preferred_element_type=jnp.float32) + ba[...]
    ou[...] = jnp.dot(U, wu[...], preferred_element_type=jnp.float32)
    ot[...] = jnp.dot(T, wt[...], preferred_element_type=jnp.float32) + bt[...]


_tc_heads = pl.pallas_call(
    _tc_heads_body,
    grid=(GRID,),
    in_specs=[_rows_spec(HH)] * 6 + [
        _full_spec(H, 4), _full_spec(1, 4),
        _full_spec(H, 1),
        _full_spec(H, 3), _full_spec(1, 3)],
    out_specs=[_rows_spec(4), _rows_spec(1), _rows_spec(3)],
    out_shape=[jax.ShapeDtypeStruct((NP, 4), jnp.float32),
               jax.ShapeDtypeStruct((NP, 1), jnp.float32),
               jax.ShapeDtypeStruct((NP, 3), jnp.float32)],
)


def kernel(x_ap, x_user, x_target, edge_index_serves, edge_index_senses, params):
    p = params
    f32 = jnp.float32

    def pad_edges(ei):
        padc = jnp.full((2, EP - E), N, jnp.int32)
        return jnp.concatenate([ei.astype(jnp.int32), padc], axis=1).reshape(
            2, NCH, 128)

    ei_sv = pad_edges(edge_index_serves)
    ei_sn = pad_edges(edge_index_senses)

    def pad_rows(x):
        return jnp.pad(x, ((0, NP - N), (0, 0)))

    xa, xu, xt = pad_rows(x_ap), pad_rows(x_user), pad_rows(x_target)

    cnt_ap_sv_p, cnt_user_p = _sc_count(ei_sv)
    cnt_ap_sn_p, cnt_tgt_p = _sc_count(ei_sn)

    b2 = lambda b: b.reshape(1, H)
    h_ap = _tc_proj(xa, p['proj_ap_W'], b2(p['proj_ap_b']),
                    xu, p['proj_user_W'], b2(p['proj_user_b']),
                    xt, p['proj_target_W'], b2(p['proj_target_b']))
    hal, hah, hul, huh, htl, hth = h_ap

    for layer in ('c1', 'c2'):
        g = lambda n: p[layer + '_' + n]
        s_sv = _sc_aggr_fwd(hal, hah, ei_sv)
        s_sn = _sc_aggr_fwd(hal, hah, ei_sn)
        s_rsv = _sc_aggr_rev(hul, huh, ei_sv)
        s_rsn = _sc_aggr_rev(htl, hth, ei_sn)
        nul, nuh = _tc_upd1(s_sv[0], s_sv[1], cnt_user_p, hul, huh,
                            g('serves_Wl'), b2(g('serves_bl')), g('serves_Wr'))
        ntl, nth = _tc_upd1(s_sn[0], s_sn[1], cnt_tgt_p, htl, hth,
                            g('senses_Wl'), b2(g('senses_bl')), g('senses_Wr'))
        nal, nah = _tc_upd2(s_rsv[0], s_rsv[1], cnt_ap_sv_p,
                            s_rsn[0], s_rsn[1], cnt_ap_sn_p, hal, hah,
                            g('rev_serves_Wl'), b2(g('rev_serves_bl')),
                            g('rev_serves_Wr'),
                            g('rev_senses_Wl'), b2(g('rev_senses_bl')),
                            g('rev_senses_Wr'))
        hal, hah, hul, huh, htl, hth = nal, nah, nul, nuh, ntl, nth

    wa = jnp.concatenate([p['head_tau_W'], p['head_x_W'][:H],
                          p['head_ytx_W'][:H], p['head_yrx_W'][:H]], axis=1)
    ba = jnp.stack([p['head_tau_b'][0], p['head_x_b'][0],
                    p['head_ytx_b'][0], p['head_yrx_b'][0]]).reshape(1, 4)
    wu = p['head_x_W'][H:]
    wt = jnp.concatenate([p['head_s_W'], p['head_ytx_W'][H:],
                          p['head_yrx_W'][H:]], axis=1)
    bt = jnp.stack([p['head_s_b'][0], jnp.zeros((), f32),
                    jnp.zeros((), f32)]).reshape(1, 3)
    A, U, T = _tc_heads(hal, hah, hul, huh, htl, hth, wa, ba, wu, wt, bt)

    tau = A[:N, 0]
    s_out = T[:N, 0]
    x_log = _sc_edge_head(jnp.ravel(A[:, 1]), jnp.ravel(U[:, 0]),
                          ei_sv).reshape(EP)[:E]
    ytx = _sc_edge_head(jnp.ravel(A[:, 2]), jnp.ravel(T[:, 1]),
                        ei_sn).reshape(EP)[:E]
    yrx = _sc_edge_head(jnp.ravel(A[:, 3]), jnp.ravel(T[:, 2]),
                        ei_sn).reshape(EP)[:E]
    return (tau, s_out, x_log, ytx, yrx)


# speedup vs baseline: 10.1412x; 1.0591x over previous
import functools

import jax
import jax.numpy as jnp
from jax import lax
from jax.experimental import pallas as pl
from jax.experimental.pallas import tpu as pltpu
from jax.experimental.pallas import tpu_sc as plsc

N = 50000
NP = 50176
E = 600000
EP = 610304
NCH = EP // 128
CPT = NCH // 16
KSB = 2
NSB = CPT // KSB
CPW = NCH // 32
KE = 8
NSE = CPW // KE
TAIL = CPW - NSE * KE
H = 64
HH = 32
NC, NS = 2, 16
TPT = NP // NS
ZROWS = 224
BR = 1024
GRID = NP // BR

_MESH = plsc.VectorSubcoreMesh(
    core_axis_name="c", subcore_axis_name="s", num_cores=NC, num_subcores=NS)
_SC_PARAMS = pltpu.CompilerParams(
    use_tc_tiling_on_sc=False, needs_layout_passes=False)


def _leaky(x):
    return jnp.where(x >= 0, x, 0.01 * x)


def _sc_count_body(ei_sv_ref, ei_sn_ref, osv_a, osv_b, osn_a, osn_b,
                   ixa, ixb, acc_a, acc_b):
    c = lax.axis_index("c")
    s = lax.axis_index("s")
    wid = s * NC + c
    t0 = wid * CPW
    ones = jnp.ones((16,), jnp.float32)

    def phase(ei_ref, out_a, out_b):
        def zero(i, _):
            zv = jnp.zeros((16,), jnp.float32)
            acc_a[pl.ds(i * 16, 16)] = zv
            acc_b[pl.ds(i * 16, 16)] = zv
            return 0
        lax.fori_loop(0, NP // 16, zero, 0)

        def count_chunks(n):
            for j in range(n):
                for k in range(8):
                    plsc.addupdate_scatter(
                        acc_a, [ixa[j, pl.ds(k * 16, 16)]], ones)
                    plsc.addupdate_scatter(
                        acc_b, [ixb[j, pl.ds(k * 16, 16)]], ones)

        def body(i, _):
            ch = t0 + i * KE
            pltpu.sync_copy(ei_ref.at[0, pl.ds(ch, KE)], ixa)
            pltpu.sync_copy(ei_ref.at[1, pl.ds(ch, KE)], ixb)
            count_chunks(KE)
            return 0
        lax.fori_loop(0, NSE, body, 0)

        ch = t0 + NSE * KE
        pltpu.sync_copy(ei_ref.at[0, pl.ds(ch, TAIL)], ixa.at[pl.ds(0, TAIL)])
        pltpu.sync_copy(ei_ref.at[1, pl.ds(ch, TAIL)], ixb.at[pl.ds(0, TAIL)])
        count_chunks(TAIL)

        pltpu.sync_copy(acc_a, out_a.at[wid])
        pltpu.sync_copy(acc_b, out_b.at[wid])

    phase(ei_sv_ref, osv_a, osv_b)
    phase(ei_sn_ref, osn_a, osn_b)


_sc_count = pl.kernel(
    _sc_count_body,
    out_type=tuple(jax.ShapeDtypeStruct((NC * NS, NP), jnp.float32)
                   for _ in range(4)),
    mesh=_MESH,
    compiler_params=_SC_PARAMS,
    scratch_types=[
        pltpu.VMEM((KE, 128), jnp.int32),
        pltpu.VMEM((KE, 128), jnp.int32),
        pltpu.VMEM((NP,), jnp.float32),
        pltpu.VMEM((NP,), jnp.float32),
    ],
)


def _sc_aggr_body(hal, hah, hul, huh, htl, hth, ei_sv_ref, ei_sn_ref,
                  svl, svh, snl, snh, rvl, rvh, rnl, rnh,
                  ixs_a, ixd_a, ixs_b, ixd_b, rows_a, rows_b, zbuf, acc,
                  sia, sib, sga, sgb, ssa, ssb):
    c = lax.axis_index("c")
    s = lax.axis_index("s")
    r0 = s * TPT
    t0 = s * CPT

    def zrow(i, _):
        zv = jnp.zeros((16,), jnp.float32)
        zbuf[i, pl.ds(0, 16)] = zv
        zbuf[i, pl.ds(16, 16)] = zv
        return 0
    lax.fori_loop(0, ZROWS, zrow, 0)

    def phase(tlo_ref, thi_ref, ei_ref, src_row, dst_row, slo_ref, shi_ref):
        for j in range(TPT // ZROWS):
            pltpu.sync_copy(zbuf, acc.at[pl.ds(r0 + j * ZROWS, ZROWS)])
        plsc.subcore_barrier()

        def fire_idx(b, ixs, ixd, sem):
            pltpu.async_copy(
                ei_ref.at[src_row, pl.ds(t0 + b * KSB, KSB)], ixs, sem)
            pltpu.async_copy(
                ei_ref.at[dst_row, pl.ds(t0 + b * KSB, KSB)], ixd, sem)

        def wait_idx(ixs, ixd, sem):
            pltpu.make_async_copy(
                ei_ref.at[src_row, pl.ds(0, KSB)], ixs, sem).wait()
            pltpu.make_async_copy(
                ei_ref.at[dst_row, pl.ds(0, KSB)], ixd, sem).wait()

        def fire_g(ixs, rows, sem):
            @pl.when(c == 0)
            def _():
                for j in range(KSB):
                    pltpu.async_copy(tlo_ref.at[ixs.at[j]], rows.at[j], sem)

            @pl.when(c == 1)
            def _():
                for j in range(KSB):
                    pltpu.async_copy(thi_ref.at[ixs.at[j]], rows.at[j], sem)

        def drain_g(ixs, rows, sem):
            for j in range(KSB):
                pltpu.make_async_copy(
                    tlo_ref.at[ixs.at[j]], rows.at[j], sem).wait()

        def fire_s(ixd, rows, sem):
            for j in range(KSB):
                pltpu.async_copy(rows.at[j], acc.at[ixd.at[j]], sem, add=True)

        def drain_s(ixd, rows, sem):
            for j in range(KSB):
                pltpu.make_async_copy(
                    rows.at[j], acc.at[ixd.at[j]], sem).wait()

        fire_idx(0, ixs_a, ixd_a, sia)
        wait_idx(ixs_a, ixd_a, sia)
        fire_g(ixs_a, rows_a, sga)

        def pair(i, _):
            b1 = 2 * i + 1
            b2 = 2 * i + 2
            fire_idx(b1, ixs_b, ixd_b, sib)
            drain_g(ixs_a, rows_a, sga)
            fire_s(ixd_a, rows_a, ssa)
            wait_idx(ixs_b, ixd_b, sib)
            fire_g(ixs_b, rows_b, sgb)
            drain_s(ixd_a, rows_a, ssa)
            fire_idx(b2, ixs_a, ixd_a, sia)
            drain_g(ixs_b, rows_b, sgb)
            fire_s(ixd_b, rows_b, ssb)
            wait_idx(ixs_a, ixd_a, sia)
            fire_g(ixs_a, rows_a, sga)
            drain_s(ixd_b, rows_b, ssb)
            return 0
        lax.fori_loop(0, (NSB - 1) // 2, pair, 0)

        drain_g(ixs_a, rows_a, sga)
        fire_s(ixd_a, rows_a, ssa)
        drain_s(ixd_a, rows_a, ssa)
        plsc.subcore_barrier()

        @pl.when(c == 0)
        def _():
            pltpu.sync_copy(acc.at[pl.ds(r0, TPT)], slo_ref.at[pl.ds(r0, TPT)])

        @pl.when(c == 1)
        def _():
            pltpu.sync_copy(acc.at[pl.ds(r0, TPT)], shi_ref.at[pl.ds(r0, TPT)])

    phase(hal, hah, ei_sv_ref, 0, 1, svl, svh)
    phase(hal, hah, ei_sn_ref, 0, 1, snl, snh)
    phase(hul, huh, ei_sv_ref, 1, 0, rvl, rvh)
    phase(htl, hth, ei_sn_ref, 1, 0, rnl, rnh)


_sc_aggr_layer = pl.kernel(
    _sc_aggr_body,
    out_type=tuple(jax.ShapeDtypeStruct((NP, HH), jnp.float32)
                   for _ in range(8)),
    mesh=_MESH,
    compiler_params=_SC_PARAMS,
    scratch_types=[
        pltpu.VMEM((KSB, 128), jnp.int32),
        pltpu.VMEM((KSB, 128), jnp.int32),
        pltpu.VMEM((KSB, 128), jnp.int32),
        pltpu.VMEM((KSB, 128), jnp.int32),
        pltpu.VMEM((KSB, 128, HH), jnp.float32),
        pltpu.VMEM((KSB, 128, HH), jnp.float32),
        pltpu.VMEM((ZROWS, HH), jnp.float32),
        pltpu.VMEM_SHARED((NP, HH), jnp.float32),
        pltpu.SemaphoreType.DMA,
        pltpu.SemaphoreType.DMA,
        pltpu.SemaphoreType.DMA,
        pltpu.SemaphoreType.DMA,
        pltpu.SemaphoreType.DMA,
        pltpu.SemaphoreType.DMA,
    ],
)


def _sc_edge_head_body(tax, tbx, taytx, tbytx, tayrx, tbyrx,
                       ei_sv_ref, ei_sn_ref, ox_ref, oytx_ref, oyrx_ref,
                       tsa, tsb, ixs, ixd, ob):
    c = lax.axis_index("c")
    s = lax.axis_index("s")
    wid = s * NC + c
    t0 = wid * CPW

    def phase(ta_ref, tb_ref, ei_ref, out_ref):
        pltpu.sync_copy(ta_ref, tsa)
        pltpu.sync_copy(tb_ref, tsb)

        def gather_chunks(n):
            for j in range(n):
                for k in range(8):
                    v = (plsc.load_gather(tsa, [ixs[j, pl.ds(k * 16, 16)]])
                         + plsc.load_gather(tsb, [ixd[j, pl.ds(k * 16, 16)]]))
                    ob[j, pl.ds(k * 16, 16)] = v

        def body(i, _):
            ch = t0 + i * KE
            pltpu.sync_copy(ei_ref.at[0, pl.ds(ch, KE)], ixs)
            pltpu.sync_copy(ei_ref.at[1, pl.ds(ch, KE)], ixd)
            gather_chunks(KE)
            pltpu.sync_copy(ob, out_ref.at[pl.ds(ch, KE)])
            return 0
        lax.fori_loop(0, NSE, body, 0)

        ch = t0 + NSE * KE
        pltpu.sync_copy(ei_ref.at[0, pl.ds(ch, TAIL)], ixs.at[pl.ds(0, TAIL)])
        pltpu.sync_copy(ei_ref.at[1, pl.ds(ch, TAIL)], ixd.at[pl.ds(0, TAIL)])
        gather_chunks(TAIL)
        pltpu.sync_copy(ob.at[pl.ds(0, TAIL)], out_ref.at[pl.ds(ch, TAIL)])

    phase(tax, tbx, ei_sv_ref, ox_ref)
    phase(taytx, tbytx, ei_sn_ref, oytx_ref)
    phase(tayrx, tbyrx, ei_sn_ref, oyrx_ref)


_sc_edge_heads = pl.kernel(
    _sc_edge_head_body,
    out_type=tuple(jax.ShapeDtypeStruct((NCH, 128), jnp.float32)
                   for _ in range(3)),
    mesh=_MESH,
    compiler_params=_SC_PARAMS,
    scratch_types=[
        pltpu.VMEM((NP,), jnp.float32),
        pltpu.VMEM((NP,), jnp.float32),
        pltpu.VMEM((KE, 128), jnp.int32),
        pltpu.VMEM((KE, 128), jnp.int32),
        pltpu.VMEM((KE, 128), jnp.float32),
    ],
)


def _tc_proj_body(xa, wa, ba, xu, wu, bu, xt, wt, bt,
                  oal, oah, oul, ouh, otl, oth):
    def one(x, w, b, ol, oh):
        h = jnp.dot(x[...], w[...], preferred_element_type=jnp.float32) + b[...]
        ol[...] = h[:, :HH]
        oh[...] = h[:, HH:]
    one(xa, wa, ba, oal, oah)
    one(xu, wu, bu, oul, ouh)
    one(xt, wt, bt, otl, oth)


def _rows_spec(w):
    return pl.BlockSpec((BR, w), lambda i: (i, 0))


def _full_spec(r, c):
    return pl.BlockSpec((r, c), lambda i: (0, 0))


def _cnt_spec():
    return pl.BlockSpec((NC * NS, BR), lambda i: (0, i))


_tc_proj = pl.pallas_call(
    _tc_proj_body,
    grid=(GRID,),
    in_specs=[_rows_spec(128), _full_spec(128, H), _full_spec(1, H)] * 3,
    out_specs=[_rows_spec(HH)] * 6,
    out_shape=[jax.ShapeDtypeStruct((NP, HH), jnp.float32)] * 6,
)


def _mean(sl, sh, cp):
    S = jnp.concatenate([sl[...], sh[...]], axis=1)
    cnt = jnp.sum(cp[...], axis=0)
    return S * (1.0 / jnp.maximum(cnt, 1.0))[:, None]


def _tc_layer_body(svl, svh, cu, hul, huh, wsv, bsv, wsvr,
                   snl, snh, ct, htl, hth, wsn, bsn, wsnr,
                   rvl, rvh, ca1, rnl, rnh, ca2, hal, hah,
                   wr1, br1, wr1r, wr2, br2, wr2r,
                   oul, ouh, otl, oth, oal, oah):
    def emit(y, ol, oh):
        y = _leaky(y)
        ol[...] = y[:, :HH]
        oh[...] = y[:, HH:]

    Xu = jnp.concatenate([hul[...], huh[...]], axis=1)
    yu = (jnp.dot(_mean(svl, svh, cu), wsv[...],
                  preferred_element_type=jnp.float32) + bsv[...]
          + jnp.dot(Xu, wsvr[...], preferred_element_type=jnp.float32))
    emit(yu, oul, ouh)

    Xt = jnp.concatenate([htl[...], hth[...]], axis=1)
    yt = (jnp.dot(_mean(snl, snh, ct), wsn[...],
                  preferred_element_type=jnp.float32) + bsn[...]
          + jnp.dot(Xt, wsnr[...], preferred_element_type=jnp.float32))
    emit(yt, otl, oth)

    Xa = jnp.concatenate([hal[...], hah[...]], axis=1)
    ya = (jnp.dot(_mean(rvl, rvh, ca1), wr1[...],
                  preferred_element_type=jnp.float32) + br1[...]
          + jnp.dot(_mean(rnl, rnh, ca2), wr2[...],
                    preferred_element_type=jnp.float32) + br2[...]
          + jnp.dot(Xa, wr1r[...] + wr2r[...],
                    preferred_element_type=jnp.float32))
    emit(ya, oal, oah)


_tc_layer = pl.pallas_call(
    _tc_layer_body,
    grid=(GRID,),
    in_specs=[_rows_spec(HH), _rows_spec(HH), _cnt_spec(),
              _rows_spec(HH), _rows_spec(HH),
              _full_spec(H, H), _full_spec(1, H), _full_spec(H, H),
              _rows_spec(HH), _rows_spec(HH), _cnt_spec(),
              _rows_spec(HH), _rows_spec(HH),
              _full_spec(H, H), _full_spec(1, H), _full_spec(H, H),
              _rows_spec(HH), _rows_spec(HH), _cnt_spec(),
              _rows_spec(HH), _rows_spec(HH), _cnt_spec(),
              _rows_spec(HH), _rows_spec(HH),
              _full_spec(H, H), _full_spec(1, H), _full_spec(H, H),
              _full_spec(H, H), _full_spec(1, H), _full_spec(H, H)],
    out_specs=[_rows_spec(HH)] * 6,
    out_shape=[jax.ShapeDtypeStruct((NP, HH), jnp.float32)] * 6,
)


def _tc_heads_body(al, ah, ul, uh, tl, th, wa, ba, wu, wt, bt,
                   oa, ou, ot):
    A = jnp.concatenate([al[...], ah[...]], axis=1)
    U = jnp.concatenate([ul[...], uh[...]], axis=1)
    T = jnp.concatenate([tl[...], th[...]], axis=1)
    oa[...] = jnp.dot(A, wa[...], preferred_element_type=jnp.float32) + ba[...]
    ou[...] = jnp.dot(U, wu[...], preferred_element_type=jnp.float32)
    ot[...] = jnp.dot(T, wt[...], preferred_element_type=jnp.float32) + bt[...]


_tc_heads = pl.pallas_call(
    _tc_heads_body,
    grid=(GRID,),
    in_specs=[_rows_spec(HH)] * 6 + [
        _full_spec(H, 4), _full_spec(1, 4),
        _full_spec(H, 1),
        _full_spec(H, 3), _full_spec(1, 3)],
    out_specs=[_rows_spec(4), _rows_spec(1), _rows_spec(3)],
    out_shape=[jax.ShapeDtypeStruct((NP, 4), jnp.float32),
               jax.ShapeDtypeStruct((NP, 1), jnp.float32),
               jax.ShapeDtypeStruct((NP, 3), jnp.float32)],
)


def kernel(x_ap, x_user, x_target, edge_index_serves, edge_index_senses, params):
    p = params
    f32 = jnp.float32

    def pad_edges(ei):
        padc = jnp.full((2, EP - E), N, jnp.int32)
        return jnp.concatenate([ei.astype(jnp.int32), padc], axis=1).reshape(
            2, NCH, 128)

    ei_sv = pad_edges(edge_index_serves)
    ei_sn = pad_edges(edge_index_senses)

    def pad_rows(x):
        return jnp.pad(x, ((0, NP - N), (0, 0)))

    xa, xu, xt = pad_rows(x_ap), pad_rows(x_user), pad_rows(x_target)

    cnt_ap_sv_p, cnt_user_p, cnt_ap_sn_p, cnt_tgt_p = _sc_count(ei_sv, ei_sn)

    b2 = lambda b: b.reshape(1, H)
    hal, hah, hul, huh, htl, hth = _tc_proj(
        xa, p['proj_ap_W'], b2(p['proj_ap_b']),
        xu, p['proj_user_W'], b2(p['proj_user_b']),
        xt, p['proj_target_W'], b2(p['proj_target_b']))

    for layer in ('c1', 'c2'):
        g = lambda n: p[layer + '_' + n]
        (svl, svh, snl, snh, rvl, rvh, rnl, rnh) = _sc_aggr_layer(
            hal, hah, hul, huh, htl, hth, ei_sv, ei_sn)
        (hul, huh, htl, hth, hal, hah) = _tc_layer(
            svl, svh, cnt_user_p, hul, huh,
            g('serves_Wl'), b2(g('serves_bl')), g('serves_Wr'),
            snl, snh, cnt_tgt_p, htl, hth,
            g('senses_Wl'), b2(g('senses_bl')), g('senses_Wr'),
            rvl, rvh, cnt_ap_sv_p, rnl, rnh, cnt_ap_sn_p, hal, hah,
            g('rev_serves_Wl'), b2(g('rev_serves_bl')), g('rev_serves_Wr'),
            g('rev_senses_Wl'), b2(g('rev_senses_bl')), g('rev_senses_Wr'))

    wa = jnp.concatenate([p['head_tau_W'], p['head_x_W'][:H],
                          p['head_ytx_W'][:H], p['head_yrx_W'][:H]], axis=1)
    ba = jnp.stack([p['head_tau_b'][0], p['head_x_b'][0],
                    p['head_ytx_b'][0], p['head_yrx_b'][0]]).reshape(1, 4)
    wu = p['head_x_W'][H:]
    wt = jnp.concatenate([p['head_s_W'], p['head_ytx_W'][H:],
                          p['head_yrx_W'][H:]], axis=1)
    bt = jnp.stack([p['head_s_b'][0], jnp.zeros((), f32),
                    jnp.zeros((), f32)]).reshape(1, 3)
    A, U, T = _tc_heads(hal, hah, hul, huh, htl, hth, wa, ba, wu, wt, bt)

    o_x, o_ytx, o_yrx = _sc_edge_heads(
        jnp.ravel(A[:, 1]), jnp.ravel(U[:, 0]),
        jnp.ravel(A[:, 2]), jnp.ravel(T[:, 1]),
        jnp.ravel(A[:, 3]), jnp.ravel(T[:, 2]),
        ei_sv, ei_sn)

    tau = A[:N, 0]
    s_out = T[:N, 0]
    x_log = o_x.reshape(EP)[:E]
    ytx = o_ytx.reshape(EP)[:E]
    yrx = o_yrx.reshape(EP)[:E]
    return (tau, s_out, x_log, ytx, yrx)


# speedup vs baseline: 11.3176x; 1.1160x over previous
import functools

import jax
import jax.numpy as jnp
from jax import lax
from jax.experimental import pallas as pl
from jax.experimental.pallas import tpu as pltpu
from jax.experimental.pallas import tpu_sc as plsc

N = 50000
NP = 50176
E = 600000
EP = 610304
NCH = EP // 128
CPT = NCH // 16
KSB = 2
NSB = CPT // KSB
CPW = NCH // 32
KE = 8
NSE = CPW // KE
TAIL = CPW - NSE * KE
H = 64
HH = 32
NC, NS = 2, 16
TPT = NP // NS
ZROWS = 112
BR = 1024
GRID = NP // BR

_MESH = plsc.VectorSubcoreMesh(
    core_axis_name="c", subcore_axis_name="s", num_cores=NC, num_subcores=NS)
_SC_PARAMS = pltpu.CompilerParams(
    use_tc_tiling_on_sc=False, needs_layout_passes=False)


def _leaky(x):
    return jnp.where(x >= 0, x, 0.01 * x)


def _sc_count_body(ei_sv_ref, ei_sn_ref, osv_a, osv_b, osn_a, osn_b,
                   ixa, ixb, acc_a, acc_b):
    c = lax.axis_index("c")
    s = lax.axis_index("s")
    wid = s * NC + c
    t0 = wid * CPW
    ones = jnp.ones((16,), jnp.float32)

    def phase(ei_ref, out_a, out_b):
        def zero(i, _):
            zv = jnp.zeros((16,), jnp.float32)
            acc_a[pl.ds(i * 16, 16)] = zv
            acc_b[pl.ds(i * 16, 16)] = zv
            return 0
        lax.fori_loop(0, NP // 16, zero, 0)

        def count_chunks(n):
            for j in range(n):
                for k in range(8):
                    plsc.addupdate_scatter(
                        acc_a, [ixa[j, pl.ds(k * 16, 16)]], ones)
                    plsc.addupdate_scatter(
                        acc_b, [ixb[j, pl.ds(k * 16, 16)]], ones)

        def body(i, _):
            ch = t0 + i * KE
            pltpu.sync_copy(ei_ref.at[0, pl.ds(ch, KE)], ixa)
            pltpu.sync_copy(ei_ref.at[1, pl.ds(ch, KE)], ixb)
            count_chunks(KE)
            return 0
        lax.fori_loop(0, NSE, body, 0)

        ch = t0 + NSE * KE
        pltpu.sync_copy(ei_ref.at[0, pl.ds(ch, TAIL)], ixa.at[pl.ds(0, TAIL)])
        pltpu.sync_copy(ei_ref.at[1, pl.ds(ch, TAIL)], ixb.at[pl.ds(0, TAIL)])
        count_chunks(TAIL)

        pltpu.sync_copy(acc_a, out_a.at[wid])
        pltpu.sync_copy(acc_b, out_b.at[wid])

    phase(ei_sv_ref, osv_a, osv_b)
    phase(ei_sn_ref, osn_a, osn_b)


_sc_count = pl.kernel(
    _sc_count_body,
    out_type=tuple(jax.ShapeDtypeStruct((NC * NS, NP), jnp.float32)
                   for _ in range(4)),
    mesh=_MESH,
    compiler_params=_SC_PARAMS,
    scratch_types=[
        pltpu.VMEM((KE, 128), jnp.int32),
        pltpu.VMEM((KE, 128), jnp.int32),
        pltpu.VMEM((NP,), jnp.float32),
        pltpu.VMEM((NP,), jnp.float32),
    ],
)


def _sc_aggr_body(hal, hah, hul, huh, htl, hth, ei_sv_ref, ei_sn_ref,
                  svl, svh, snl, snh, rvl, rvh, rnl, rnh,
                  ixs_a, ixd_a, ixs_b, ixd_b, ixs_c, ixd_c,
                  rows_a, rows_b, rows_c, zbuf, acc,
                  sia, sib, sic, sga, sgb, sgc, ssa, ssb, ssc):
    c = lax.axis_index("c")
    s = lax.axis_index("s")
    r0 = s * TPT
    t0 = s * CPT

    def zrow(i, _):
        zv = jnp.zeros((16,), jnp.float32)
        zbuf[i, pl.ds(0, 16)] = zv
        zbuf[i, pl.ds(16, 16)] = zv
        return 0
    lax.fori_loop(0, ZROWS, zrow, 0)

    def phase(tlo_ref, thi_ref, ei_ref, src_row, dst_row, slo_ref, shi_ref):
        for j in range(TPT // ZROWS):
            pltpu.sync_copy(zbuf, acc.at[pl.ds(r0 + j * ZROWS, ZROWS)])
        plsc.subcore_barrier()

        def fire_idx(b, ixs, ixd, sem):
            pltpu.async_copy(
                ei_ref.at[src_row, pl.ds(t0 + b * KSB, KSB)], ixs, sem)
            pltpu.async_copy(
                ei_ref.at[dst_row, pl.ds(t0 + b * KSB, KSB)], ixd, sem)

        def wait_idx(ixs, ixd, sem):
            pltpu.make_async_copy(
                ei_ref.at[src_row, pl.ds(0, KSB)], ixs, sem).wait()
            pltpu.make_async_copy(
                ei_ref.at[dst_row, pl.ds(0, KSB)], ixd, sem).wait()

        def fire_g(ixs, rows, sem):
            @pl.when(c == 0)
            def _():
                for j in range(KSB):
                    pltpu.async_copy(tlo_ref.at[ixs.at[j]], rows.at[j], sem)

            @pl.when(c == 1)
            def _():
                for j in range(KSB):
                    pltpu.async_copy(thi_ref.at[ixs.at[j]], rows.at[j], sem)

        def drain_g(ixs, rows, sem):
            for j in range(KSB):
                pltpu.make_async_copy(
                    tlo_ref.at[ixs.at[j]], rows.at[j], sem).wait()

        def fire_s(ixd, rows, sem):
            for j in range(KSB):
                pltpu.async_copy(rows.at[j], acc.at[ixd.at[j]], sem, add=True)

        def drain_s(ixd, rows, sem):
            for j in range(KSB):
                pltpu.make_async_copy(
                    rows.at[j], acc.at[ixd.at[j]], sem).wait()

        fire_idx(0, ixs_a, ixd_a, sia)
        fire_idx(1, ixs_b, ixd_b, sib)
        wait_idx(ixs_a, ixd_a, sia)
        fire_g(ixs_a, rows_a, sga)
        wait_idx(ixs_b, ixd_b, sib)
        fire_g(ixs_b, rows_b, sgb)

        def triple(i, _):
            b = 3 * i
            fire_idx(b + 2, ixs_c, ixd_c, sic)
            drain_g(ixs_a, rows_a, sga)
            fire_s(ixd_a, rows_a, ssa)
            wait_idx(ixs_c, ixd_c, sic)
            fire_g(ixs_c, rows_c, sgc)
            drain_s(ixd_a, rows_a, ssa)
            fire_idx(b + 3, ixs_a, ixd_a, sia)
            drain_g(ixs_b, rows_b, sgb)
            fire_s(ixd_b, rows_b, ssb)
            wait_idx(ixs_a, ixd_a, sia)
            fire_g(ixs_a, rows_a, sga)
            drain_s(ixd_b, rows_b, ssb)
            fire_idx(b + 4, ixs_b, ixd_b, sib)
            drain_g(ixs_c, rows_c, sgc)
            fire_s(ixd_c, rows_c, ssc)
            wait_idx(ixs_b, ixd_b, sib)
            fire_g(ixs_b, rows_b, sgb)
            drain_s(ixd_c, rows_c, ssc)
            return 0
        lax.fori_loop(0, (NSB - 2) // 3, triple, 0)

        drain_g(ixs_a, rows_a, sga)
        fire_s(ixd_a, rows_a, ssa)
        drain_s(ixd_a, rows_a, ssa)
        drain_g(ixs_b, rows_b, sgb)
        fire_s(ixd_b, rows_b, ssb)
        drain_s(ixd_b, rows_b, ssb)
        plsc.subcore_barrier()

        @pl.when(c == 0)
        def _():
            pltpu.sync_copy(acc.at[pl.ds(r0, TPT)], slo_ref.at[pl.ds(r0, TPT)])

        @pl.when(c == 1)
        def _():
            pltpu.sync_copy(acc.at[pl.ds(r0, TPT)], shi_ref.at[pl.ds(r0, TPT)])

    phase(hal, hah, ei_sv_ref, 0, 1, svl, svh)
    phase(hal, hah, ei_sn_ref, 0, 1, snl, snh)
    phase(hul, huh, ei_sv_ref, 1, 0, rvl, rvh)
    phase(htl, hth, ei_sn_ref, 1, 0, rnl, rnh)


_sc_aggr_layer = pl.kernel(
    _sc_aggr_body,
    out_type=tuple(jax.ShapeDtypeStruct((NP, HH), jnp.float32)
                   for _ in range(8)),
    mesh=_MESH,
    compiler_params=_SC_PARAMS,
    scratch_types=[
        pltpu.VMEM((KSB, 128), jnp.int32),
        pltpu.VMEM((KSB, 128), jnp.int32),
        pltpu.VMEM((KSB, 128), jnp.int32),
        pltpu.VMEM((KSB, 128), jnp.int32),
        pltpu.VMEM((KSB, 128), jnp.int32),
        pltpu.VMEM((KSB, 128), jnp.int32),
        pltpu.VMEM((KSB, 128, HH), jnp.float32),
        pltpu.VMEM((KSB, 128, HH), jnp.float32),
        pltpu.VMEM((KSB, 128, HH), jnp.float32),
        pltpu.VMEM((ZROWS, HH), jnp.float32),
        pltpu.VMEM_SHARED((NP, HH), jnp.float32),
        pltpu.SemaphoreType.DMA,
        pltpu.SemaphoreType.DMA,
        pltpu.SemaphoreType.DMA,
        pltpu.SemaphoreType.DMA,
        pltpu.SemaphoreType.DMA,
        pltpu.SemaphoreType.DMA,
        pltpu.SemaphoreType.DMA,
        pltpu.SemaphoreType.DMA,
        pltpu.SemaphoreType.DMA,
    ],
)


def _sc_edge_head_body(tax, tbx, taytx, tbytx, tayrx, tbyrx,
                       ei_sv_ref, ei_sn_ref, ox_ref, oytx_ref, oyrx_ref,
                       tsa, tsb, ixs, ixd, ob):
    c = lax.axis_index("c")
    s = lax.axis_index("s")
    wid = s * NC + c
    t0 = wid * CPW

    def phase(ta_ref, tb_ref, ei_ref, out_ref):
        pltpu.sync_copy(ta_ref, tsa)
        pltpu.sync_copy(tb_ref, tsb)

        def gather_chunks(n):
            for j in range(n):
                for k in range(8):
                    v = (plsc.load_gather(tsa, [ixs[j, pl.ds(k * 16, 16)]])
                         + plsc.load_gather(tsb, [ixd[j, pl.ds(k * 16, 16)]]))
                    ob[j, pl.ds(k * 16, 16)] = v

        def body(i, _):
            ch = t0 + i * KE
            pltpu.sync_copy(ei_ref.at[0, pl.ds(ch, KE)], ixs)
            pltpu.sync_copy(ei_ref.at[1, pl.ds(ch, KE)], ixd)
            gather_chunks(KE)
            pltpu.sync_copy(ob, out_ref.at[pl.ds(ch, KE)])
            return 0
        lax.fori_loop(0, NSE, body, 0)

        ch = t0 + NSE * KE
        pltpu.sync_copy(ei_ref.at[0, pl.ds(ch, TAIL)], ixs.at[pl.ds(0, TAIL)])
        pltpu.sync_copy(ei_ref.at[1, pl.ds(ch, TAIL)], ixd.at[pl.ds(0, TAIL)])
        gather_chunks(TAIL)
        pltpu.sync_copy(ob.at[pl.ds(0, TAIL)], out_ref.at[pl.ds(ch, TAIL)])

    phase(tax, tbx, ei_sv_ref, ox_ref)
    phase(taytx, tbytx, ei_sn_ref, oytx_ref)
    phase(tayrx, tbyrx, ei_sn_ref, oyrx_ref)


_sc_edge_heads = pl.kernel(
    _sc_edge_head_body,
    out_type=tuple(jax.ShapeDtypeStruct((NCH, 128), jnp.float32)
                   for _ in range(3)),
    mesh=_MESH,
    compiler_params=_SC_PARAMS,
    scratch_types=[
        pltpu.VMEM((NP,), jnp.float32),
        pltpu.VMEM((NP,), jnp.float32),
        pltpu.VMEM((KE, 128), jnp.int32),
        pltpu.VMEM((KE, 128), jnp.int32),
        pltpu.VMEM((KE, 128), jnp.float32),
    ],
)


def _tc_proj_body(xa, wa, ba, xu, wu, bu, xt, wt, bt,
                  oal, oah, oul, ouh, otl, oth):
    def one(x, w, b, ol, oh):
        h = jnp.dot(x[...], w[...], preferred_element_type=jnp.float32) + b[...]
        ol[...] = h[:, :HH]
        oh[...] = h[:, HH:]
    one(xa, wa, ba, oal, oah)
    one(xu, wu, bu, oul, ouh)
    one(xt, wt, bt, otl, oth)


def _rows_spec(w):
    return pl.BlockSpec((BR, w), lambda i: (i, 0))


def _full_spec(r, c):
    return pl.BlockSpec((r, c), lambda i: (0, 0))


def _cnt_spec():
    return pl.BlockSpec((NC * NS, BR), lambda i: (0, i))


_tc_proj = pl.pallas_call(
    _tc_proj_body,
    grid=(GRID,),
    in_specs=[_rows_spec(128), _full_spec(128, H), _full_spec(1, H)] * 3,
    out_specs=[_rows_spec(HH)] * 6,
    out_shape=[jax.ShapeDtypeStruct((NP, HH), jnp.float32)] * 6,
)


def _mean(sl, sh, cp):
    S = jnp.concatenate([sl[...], sh[...]], axis=1)
    cnt = jnp.sum(cp[...], axis=0)
    return S * (1.0 / jnp.maximum(cnt, 1.0))[:, None]


def _tc_layer_body(svl, svh, cu, hul, huh, wsv, bsv, wsvr,
                   snl, snh, ct, htl, hth, wsn, bsn, wsnr,
                   rvl, rvh, ca1, rnl, rnh, ca2, hal, hah,
                   wr1, br1, wr1r, wr2, br2, wr2r,
                   oul, ouh, otl, oth, oal, oah):
    def emit(y, ol, oh):
        y = _leaky(y)
        ol[...] = y[:, :HH]
        oh[...] = y[:, HH:]

    Xu = jnp.concatenate([hul[...], huh[...]], axis=1)
    yu = (jnp.dot(_mean(svl, svh, cu), wsv[...],
                  preferred_element_type=jnp.float32) + bsv[...]
          + jnp.dot(Xu, wsvr[...], preferred_element_type=jnp.float32))
    emit(yu, oul, ouh)

    Xt = jnp.concatenate([htl[...], hth[...]], axis=1)
    yt = (jnp.dot(_mean(snl, snh, ct), wsn[...],
                  preferred_element_type=jnp.float32) + bsn[...]
          + jnp.dot(Xt, wsnr[...], preferred_element_type=jnp.float32))
    emit(yt, otl, oth)

    Xa = jnp.concatenate([hal[...], hah[...]], axis=1)
    ya = (jnp.dot(_mean(rvl, rvh, ca1), wr1[...],
                  preferred_element_type=jnp.float32) + br1[...]
          + jnp.dot(_mean(rnl, rnh, ca2), wr2[...],
                    preferred_element_type=jnp.float32) + br2[...]
          + jnp.dot(Xa, wr1r[...] + wr2r[...],
                    preferred_element_type=jnp.float32))
    emit(ya, oal, oah)


_tc_layer = pl.pallas_call(
    _tc_layer_body,
    grid=(GRID,),
    in_specs=[_rows_spec(HH), _rows_spec(HH), _cnt_spec(),
              _rows_spec(HH), _rows_spec(HH),
              _full_spec(H, H), _full_spec(1, H), _full_spec(H, H),
              _rows_spec(HH), _rows_spec(HH), _cnt_spec(),
              _rows_spec(HH), _rows_spec(HH),
              _full_spec(H, H), _full_spec(1, H), _full_spec(H, H),
              _rows_spec(HH), _rows_spec(HH), _cnt_spec(),
              _rows_spec(HH), _rows_spec(HH), _cnt_spec(),
              _rows_spec(HH), _rows_spec(HH),
              _full_spec(H, H), _full_spec(1, H), _full_spec(H, H),
              _full_spec(H, H), _full_spec(1, H), _full_spec(H, H)],
    out_specs=[_rows_spec(HH)] * 6,
    out_shape=[jax.ShapeDtypeStruct((NP, HH), jnp.float32)] * 6,
)


def _tc_heads_body(al, ah, ul, uh, tl, th, wa, ba, wu, wt, bt,
                   oa, ou, ot):
    A = jnp.concatenate([al[...], ah[...]], axis=1)
    U = jnp.concatenate([ul[...], uh[...]], axis=1)
    T = jnp.concatenate([tl[...], th[...]], axis=1)
    oa[...] = jnp.dot(A, wa[...], preferred_element_type=jnp.float32) + ba[...]
    ou[...] = jnp.dot(U, wu[...], preferred_element_type=jnp.float32)
    ot[...] = jnp.dot(T, wt[...], preferred_element_type=jnp.float32) + bt[...]


_tc_heads = pl.pallas_call(
    _tc_heads_body,
    grid=(GRID,),
    in_specs=[_rows_spec(HH)] * 6 + [
        _full_spec(H, 4), _full_spec(1, 4),
        _full_spec(H, 1),
        _full_spec(H, 3), _full_spec(1, 3)],
    out_specs=[_rows_spec(4), _rows_spec(1), _rows_spec(3)],
    out_shape=[jax.ShapeDtypeStruct((NP, 4), jnp.float32),
               jax.ShapeDtypeStruct((NP, 1), jnp.float32),
               jax.ShapeDtypeStruct((NP, 3), jnp.float32)],
)


def kernel(x_ap, x_user, x_target, edge_index_serves, edge_index_senses, params):
    p = params
    f32 = jnp.float32

    def pad_edges(ei):
        padc = jnp.full((2, EP - E), N, jnp.int32)
        return jnp.concatenate([ei.astype(jnp.int32), padc], axis=1).reshape(
            2, NCH, 128)

    ei_sv = pad_edges(edge_index_serves)
    ei_sn = pad_edges(edge_index_senses)

    def pad_rows(x):
        return jnp.pad(x, ((0, NP - N), (0, 0)))

    xa, xu, xt = pad_rows(x_ap), pad_rows(x_user), pad_rows(x_target)

    cnt_ap_sv_p, cnt_user_p, cnt_ap_sn_p, cnt_tgt_p = _sc_count(ei_sv, ei_sn)

    b2 = lambda b: b.reshape(1, H)
    hal, hah, hul, huh, htl, hth = _tc_proj(
        xa, p['proj_ap_W'], b2(p['proj_ap_b']),
        xu, p['proj_user_W'], b2(p['proj_user_b']),
        xt, p['proj_target_W'], b2(p['proj_target_b']))

    for layer in ('c1', 'c2'):
        g = lambda n: p[layer + '_' + n]
        (svl, svh, snl, snh, rvl, rvh, rnl, rnh) = _sc_aggr_layer(
            hal, hah, hul, huh, htl, hth, ei_sv, ei_sn)
        (hul, huh, htl, hth, hal, hah) = _tc_layer(
            svl, svh, cnt_user_p, hul, huh,
            g('serves_Wl'), b2(g('serves_bl')), g('serves_Wr'),
            snl, snh, cnt_tgt_p, htl, hth,
            g('senses_Wl'), b2(g('senses_bl')), g('senses_Wr'),
            rvl, rvh, cnt_ap_sv_p, rnl, rnh, cnt_ap_sn_p, hal, hah,
            g('rev_serves_Wl'), b2(g('rev_serves_bl')), g('rev_serves_Wr'),
            g('rev_senses_Wl'), b2(g('rev_senses_bl')), g('rev_senses_Wr'))

    wa = jnp.concatenate([p['head_tau_W'], p['head_x_W'][:H],
                          p['head_ytx_W'][:H], p['head_yrx_W'][:H]], axis=1)
    ba = jnp.stack([p['head_tau_b'][0], p['head_x_b'][0],
                    p['head_ytx_b'][0], p['head_yrx_b'][0]]).reshape(1, 4)
    wu = p['head_x_W'][H:]
    wt = jnp.concatenate([p['head_s_W'], p['head_ytx_W'][H:],
                          p['head_yrx_W'][H:]], axis=1)
    bt = jnp.stack([p['head_s_b'][0], jnp.zeros((), f32),
                    jnp.zeros((), f32)]).reshape(1, 3)
    A, U, T = _tc_heads(hal, hah, hul, huh, htl, hth, wa, ba, wu, wt, bt)

    o_x, o_ytx, o_yrx = _sc_edge_heads(
        jnp.ravel(A[:, 1]), jnp.ravel(U[:, 0]),
        jnp.ravel(A[:, 2]), jnp.ravel(T[:, 1]),
        jnp.ravel(A[:, 3]), jnp.ravel(T[:, 2]),
        ei_sv, ei_sn)

    tau = A[:N, 0]
    s_out = T[:N, 0]
    x_log = o_x.reshape(EP)[:E]
    ytx = o_ytx.reshape(EP)[:E]
    yrx = o_yrx.reshape(EP)[:E]
    return (tau, s_out, x_log, ytx, yrx)
